# baseline (speedup 1.0000x reference)
import functools

import jax
import jax.numpy as jnp
from jax import lax
from jax.experimental import pallas as pl
from jax.experimental.pallas import tpu as pltpu
from jax.experimental.pallas import tpu_sc as plsc

_NUM_CORES = 2
_NUM_SUBCORES = 16
_NW = _NUM_CORES * _NUM_SUBCORES
_L = 16
_HC = 2


@functools.lru_cache(maxsize=None)
def _make_gather(BQ: int, H: int, D: int):
    assert BQ % _NW == 0
    bpt = BQ // _NW
    assert bpt % 128 == 0 and D % _L == 0 and H % _HC == 0
    npt = bpt * H
    nchunk = H // _HC
    rows = _HC * bpt
    assert nchunk % 2 == 0 and nchunk >= 6
    mesh = plsc.VectorSubcoreMesh(core_axis_name="c", subcore_axis_name="s")

    @functools.partial(
        pl.kernel,
        mesh=mesh,
        out_type=jax.ShapeDtypeStruct((H, D, BQ), jnp.float32),
        scratch_types=[
            pltpu.VMEM((npt,), jnp.int32),
            [pltpu.VMEM((rows,), jnp.int32) for _ in range(2)],
            [pltpu.VMEM((rows, 2 * D), jnp.float32) for _ in range(2)],
            [pltpu.VMEM((_HC, D, bpt), jnp.float32) for _ in range(2)],
            pltpu.SemaphoreType.DMA,
            pltpu.SemaphoreType.DMA,
        ],
        compiler_params=pltpu.CompilerParams(use_tc_tiling_on_sc=True,
                                             needs_layout_passes=False),
    )
    def gather(idx_hbm, table_hbm, out_hbm, idx_v, cidx, bufa, bufb,
               gsem, wsem):
        wid = lax.axis_index("s") * _NUM_CORES + lax.axis_index("c")
        b0 = wid * bpt
        pltpu.sync_copy(idx_hbm.at[pl.ds(b0 * H, npt)], idx_v)

        iota = jnp.arange(_L, dtype=jnp.int32)
        posv = [(k * _L + iota) * H for k in range(bpt // _L)]
        rowv = [[hh * bpt + k * _L + iota for k in range(bpt // _L)]
                for hh in range(_HC)]

        def build_cidx(c, ci):
            h0 = c * _HC
            for hh in range(_HC):
                for k in range(bpt // _L):
                    v = plsc.load_gather(idx_v, [posv[k] + (h0 + hh)])
                    ci.at[pl.ds(hh * bpt + k * _L, _L)][...] = v

        def g_start(ci, a):
            pltpu.async_copy(table_hbm.at[ci], a, gsem)

        def g_wait(ci, a):
            pltpu.make_async_copy(table_hbm.at[ci], a, gsem).wait()

        def w_start(c, b):
            pltpu.async_copy(
                b, out_hbm.at[pl.ds(c * _HC, _HC), :, pl.ds(b0, bpt)], wsem)

        def w_wait(b):
            pltpu.make_async_copy(
                b, out_hbm.at[pl.ds(0, _HC), :, pl.ds(b0, bpt)], wsem).wait()

        def transpose(a, b):
            def per_d(d, carry):
                dv = jnp.zeros((_L,), jnp.int32) + d
                for hh in range(_HC):
                    for k in range(bpt // _L):
                        v = plsc.load_gather(a, [rowv[hh][k], dv])
                        b.at[hh, d, pl.ds(k * _L, _L)][...] = v
                return carry
            lax.fori_loop(0, D, per_d, 0)

        for p in range(2):
            build_cidx(p, cidx[p])
            g_start(cidx[p], bufa[p])
        for p in range(2):
            g_wait(cidx[p], bufa[p])
            transpose(bufa[p], bufb[p])
            build_cidx(2 + p, cidx[p])
            g_start(cidx[p], bufa[p])
            w_start(p, bufb[p])

        def body(j, carry):
            for p in range(2):
                c = 2 * j + p
                g_wait(cidx[p], bufa[p])
                w_wait(bufb[p])
                transpose(bufa[p], bufb[p])
                build_cidx(c + 2, cidx[p])
                g_start(cidx[p], bufa[p])
                w_start(c, bufb[p])
            return carry

        lax.fori_loop(1, nchunk // 2 - 1, body, 0)

        for p in range(2):
            c = nchunk - 2 + p
            g_wait(cidx[p], bufa[p])
            w_wait(bufb[p])
            transpose(bufa[p], bufb[p])
            w_start(c, bufb[p])
        for p in range(2):
            w_wait(bufb[p])

    return gather


def kernel(x, table):
    bq, hist = x.shape
    d = table.shape[1]
    idx = x.reshape(bq * hist).astype(jnp.int32)
    table_pad = jnp.pad(table, ((0, 0), (0, d)))
    out = _make_gather(bq, hist, d)(idx, table_pad)
    return jnp.transpose(out, (2, 0, 1))


# speedup vs baseline: 1.2164x; 1.2164x over previous
import functools

import jax
import jax.numpy as jnp
from jax import lax
from jax.experimental import pallas as pl
from jax.experimental.pallas import tpu as pltpu
from jax.experimental.pallas import tpu_sc as plsc

_NUM_CORES = 2
_NUM_SUBCORES = 16
_NW = _NUM_CORES * _NUM_SUBCORES
_L = 16
_HC = 2


@functools.lru_cache(maxsize=None)
def _make_gather(BQ: int, H: int, D: int):
    assert BQ % _NW == 0
    bpt = BQ // _NW
    assert bpt % 128 == 0 and D % _L == 0 and H % _HC == 0
    npt = bpt * H
    nchunk = H // _HC
    rows = _HC * bpt
    assert nchunk % 2 == 0 and nchunk >= 6
    mesh = plsc.VectorSubcoreMesh(core_axis_name="c", subcore_axis_name="s")

    @functools.partial(
        pl.kernel,
        mesh=mesh,
        out_type=jax.ShapeDtypeStruct((H, D, BQ), jnp.float32),
        scratch_types=[
            pltpu.VMEM((npt,), jnp.int32),
            [pltpu.VMEM((rows,), jnp.int32) for _ in range(2)],
            [pltpu.VMEM((rows, 2 * D), jnp.float32) for _ in range(2)],
            [pltpu.VMEM((_HC, D, bpt), jnp.float32) for _ in range(2)],
            pltpu.SemaphoreType.DMA,
            pltpu.SemaphoreType.DMA,
        ],
        compiler_params=pltpu.CompilerParams(use_tc_tiling_on_sc=True,
                                             needs_layout_passes=False),
    )
    def gather(idx_hbm, table_hbm, out_hbm, idx_v, cidx, bufa, bufb,
               gsem, wsem):
        wid = lax.axis_index("s") * _NUM_CORES + lax.axis_index("c")
        b0 = wid * bpt
        pltpu.sync_copy(idx_hbm.at[pl.ds(b0 * H, npt)], idx_v)

        iota = jnp.arange(_L, dtype=jnp.int32)
        posv = [(k * _L + iota) * H for k in range(bpt // _L)]
        rowv = [[hh * bpt + k * _L + iota for k in range(bpt // _L)]
                for hh in range(_HC)]

        def build_cidx(c, ci):
            h0 = c * _HC
            for hh in range(_HC):
                for k in range(bpt // _L):
                    v = plsc.load_gather(idx_v, [posv[k] + (h0 + hh)])
                    ci.at[pl.ds(hh * bpt + k * _L, _L)][...] = v

        def g_start(ci, a):
            pltpu.async_copy(table_hbm.at[ci], a, gsem)

        def g_wait(ci, a):
            pltpu.make_async_copy(table_hbm.at[ci], a, gsem).wait()

        def w_start(c, b):
            pltpu.async_copy(
                b, out_hbm.at[pl.ds(c * _HC, _HC), :, pl.ds(b0, bpt)], wsem)

        def w_wait(b):
            pltpu.make_async_copy(
                b, out_hbm.at[pl.ds(0, _HC), :, pl.ds(b0, bpt)], wsem).wait()

        dvecs = [iota + _L * k for k in range(D // _L)]
        zeros = jnp.zeros((_L,), jnp.int32)

        def transpose(a, b):
            for hh in range(_HC):
                hv = zeros + hh

                def per_b(bp, carry):
                    bv = zeros + bp
                    for k in range(D // _L):
                        v = a.at[hh * bpt + bp, pl.ds(k * _L, _L)][...]
                        plsc.store_scatter(b, [hv, dvecs[k], bv], v)
                    return carry
                lax.fori_loop(0, bpt, per_b, 0)

        for p in range(2):
            build_cidx(p, cidx[p])
            g_start(cidx[p], bufa[p])
        for p in range(2):
            g_wait(cidx[p], bufa[p])
            transpose(bufa[p], bufb[p])
            build_cidx(2 + p, cidx[p])
            g_start(cidx[p], bufa[p])
            w_start(p, bufb[p])

        def body(j, carry):
            for p in range(2):
                c = 2 * j + p
                g_wait(cidx[p], bufa[p])
                w_wait(bufb[p])
                transpose(bufa[p], bufb[p])
                build_cidx(c + 2, cidx[p])
                g_start(cidx[p], bufa[p])
                w_start(c, bufb[p])
            return carry

        lax.fori_loop(1, nchunk // 2 - 1, body, 0)

        for p in range(2):
            c = nchunk - 2 + p
            g_wait(cidx[p], bufa[p])
            w_wait(bufb[p])
            transpose(bufa[p], bufb[p])
            w_start(c, bufb[p])
        for p in range(2):
            w_wait(bufb[p])

    return gather


def kernel(x, table):
    bq, hist = x.shape
    d = table.shape[1]
    idx = x.reshape(bq * hist).astype(jnp.int32)
    table_pad = jnp.pad(table, ((0, 0), (0, d)))
    out = _make_gather(bq, hist, d)(idx, table_pad)
    return jnp.transpose(out, (2, 0, 1))
